# baseline (speedup 1.0000x reference)
import jax
import jax.numpy as jnp
from jax import lax
from jax.experimental import pallas as pl
from jax.experimental.pallas import tpu as pltpu
from jax.experimental.pallas import tpu_sc as plsc

B = 128
HIDDEN = 1024
NUM_EXPERTS = 64
TOP_K = 8
INTER = 512
NORM_MIN = 1e-12
E_PER = 2

NC = 2
NS = 16
NW = NC * NS
TPW = B // NW
NCHUNK = NUM_EXPERTS // 16


def _logits_kernel(x_ref, rw_ref, logits_ref, probs_ref):
    logits = jnp.dot(x_ref[...], rw_ref[...].T,
                     preferred_element_type=jnp.float32)
    logits_ref[...] = logits
    probs_ref[...] = jax.nn.softmax(logits, axis=-1)


def _lane():
    return lax.iota(jnp.int32, 16)


def _shuf(x, k):
    idx = jnp.bitwise_and(_lane() + k, 15)
    return lax.gather(x, idx[:, None],
                      lax.GatherDimensionNumbers(offset_dims=(),
                                                 collapsed_slice_dims=(0,),
                                                 start_index_map=(0,)),
                      slice_sizes=(1,),
                      mode=lax.GatherScatterMode.PROMISE_IN_BOUNDS)


def _splat_reduce(x, op):
    for k in (8, 4, 2, 1):
        x = op(x, _shuf(x, k))
    return x


def _router_sc(probs_hbm, bias_hbm, comb_hbm, lg_v, bias_v, comb_v):
    wid = lax.axis_index("s") * NC + lax.axis_index("c")
    base = wid * TPW
    pltpu.sync_copy(probs_hbm.at[pl.ds(base, TPW)], lg_v)
    pltpu.sync_copy(bias_hbm, bias_v)
    lane = _lane()
    bias = [bias_v[0, pl.ds(c * 16, 16)] for c in range(NCHUNK)]
    for t in range(TPW):
        p = [lg_v[t, pl.ds(c * 16, 16)] for c in range(NCHUNK)]
        work = [pc + bc for pc, bc in zip(p, bias)]
        comb = [jnp.zeros((16,), jnp.float32) for _ in range(NCHUNK)]
        for _ in range(TOP_K):
            mk = _splat_reduce(
                jnp.maximum(jnp.maximum(work[0], work[1]),
                            jnp.maximum(work[2], work[3])), jnp.maximum)
            cand = [jnp.where(work[c] == mk, lane + 16 * c, NUM_EXPERTS)
                    for c in range(NCHUNK)]
            first = _splat_reduce(
                jnp.minimum(jnp.minimum(cand[0], cand[1]),
                            jnp.minimum(cand[2], cand[3])), jnp.minimum)
            for c in range(NCHUNK):
                sel = (lane + 16 * c) == first
                comb[c] = jnp.where(sel, comb[c] + p[c], comb[c])
                work[c] = jnp.where(sel, jnp.full((16,), -jnp.inf,
                                                  jnp.float32), work[c])
        tot = _splat_reduce(comb[0] + comb[1] + comb[2] + comb[3], jnp.add)
        invw = 1.0 / jnp.maximum(tot, NORM_MIN)
        for c in range(NCHUNK):
            comb_v[t, pl.ds(c * 16, 16)] = comb[c] * invw
    pltpu.sync_copy(comb_v, comb_hbm.at[pl.ds(base, TPW)])


def _expert_kernel(x_ref, comb_in_ref, gate_ref, up_ref, down_ref,
                   out_ref, xb_ref):
    i = pl.program_id(0)

    @pl.when(i == 0)
    def _init():
        out_ref[...] = jnp.zeros_like(out_ref)
        xb_ref[...] = x_ref[...].astype(jnp.bfloat16)

    xb = xb_ref[...]
    ecol = jax.lax.broadcasted_iota(jnp.int32, (B, NUM_EXPERTS), 1)
    comb = comb_in_ref[...]
    for j in range(E_PER):
        g = jnp.dot(xb, gate_ref[j].astype(jnp.bfloat16),
                    preferred_element_type=jnp.float32)
        u = jnp.dot(xb, up_ref[j].astype(jnp.bfloat16),
                    preferred_element_type=jnp.float32)
        w = jnp.sum(jnp.where(ecol == i * E_PER + j, comb, 0.0),
                    axis=-1, keepdims=True)
        hw = (jax.nn.silu(g) * u * w).astype(jnp.bfloat16)
        out_ref[...] += jnp.dot(hw, down_ref[j].astype(jnp.bfloat16),
                                preferred_element_type=jnp.float32)


@jax.jit
def kernel(hidden_states, router_weight, e_bias, gate_w, up_w, down_w):
    shape = hidden_states.shape
    x = hidden_states.reshape(-1, HIDDEN)

    logits, probs = pl.pallas_call(
        _logits_kernel,
        out_shape=[jax.ShapeDtypeStruct((B, NUM_EXPERTS), jnp.float32),
                   jax.ShapeDtypeStruct((B, NUM_EXPERTS), jnp.float32)],
    )(x, router_weight)

    router = pl.kernel(
        _router_sc,
        out_type=jax.ShapeDtypeStruct((B, NUM_EXPERTS), jnp.float32),
        mesh=plsc.VectorSubcoreMesh(core_axis_name="c", subcore_axis_name="s"),
        scratch_types=[
            pltpu.VMEM((TPW, NUM_EXPERTS), jnp.float32),
            pltpu.VMEM((1, NUM_EXPERTS), jnp.float32),
            pltpu.VMEM((TPW, NUM_EXPERTS), jnp.float32),
        ],
    )
    comb = router(probs, e_bias)

    out = pl.pallas_call(
        _expert_kernel,
        grid=(NUM_EXPERTS // E_PER,),
        in_specs=[
            pl.BlockSpec((B, HIDDEN), lambda i: (0, 0)),
            pl.BlockSpec((B, NUM_EXPERTS), lambda i: (0, 0)),
            pl.BlockSpec((E_PER, HIDDEN, INTER), lambda i: (i, 0, 0)),
            pl.BlockSpec((E_PER, HIDDEN, INTER), lambda i: (i, 0, 0)),
            pl.BlockSpec((E_PER, INTER, HIDDEN), lambda i: (i, 0, 0)),
        ],
        out_specs=pl.BlockSpec((B, HIDDEN), lambda i: (0, 0)),
        out_shape=jax.ShapeDtypeStruct((B, HIDDEN), jnp.float32),
        scratch_shapes=[pltpu.VMEM((B, HIDDEN), jnp.bfloat16)],
    )(x, comb, gate_w, up_w, down_w)
    return out.reshape(shape), logits
